# baseline (speedup 1.0000x reference)
import jax
import jax.numpy as jnp
from jax import lax
from jax.experimental import pallas as pl
from jax.experimental.pallas import tpu as pltpu
from jax.experimental.pallas import tpu_sc as plsc

H = 1024
F = 4096
E = 8
NT = 4096
BT = 128
NB = NT // BT + E
NPAD = NB * BT
LANES = 128

NW = 32
TPW = NT // NW
RC = 32
NCHUNK = TPW // RC


def _router_body(x_ref, gw_ref, ids_ref):
    xb = x_ref[...]
    logits = lax.dot_general(xb, gw_ref[...], (((1,), (1,)), ((), ())),
                             preferred_element_type=jnp.float32)
    lane = lax.broadcasted_iota(jnp.int32, logits.shape, 1)
    masked = jnp.where(lane < E, logits, -jnp.inf)
    mx = jnp.max(masked, axis=1, keepdims=True)
    cand = jnp.where(masked == mx, lane, LANES)
    ids_ref[...] = jnp.min(cand, axis=1).astype(jnp.int32)


def _router(x2d, gw_pad):
    brt = 512
    return pl.pallas_call(
        _router_body,
        grid=(NT // brt,),
        in_specs=[
            pl.BlockSpec((brt, H), lambda b: (b, 0)),
            pl.BlockSpec((LANES, H), lambda b: (0, 0)),
        ],
        out_specs=pl.BlockSpec((brt,), lambda b: (b,)),
        out_shape=jax.ShapeDtypeStruct((NT,), jnp.int32),
    )(x2d, gw_pad)


def _dispatch_body(ids_ref, dest_ref, gs_ref, bexp_ref, meta_ref):
    ids = ids_ref[...]
    rows, cols = ids.shape
    r1 = lax.broadcasted_iota(jnp.int32, (cols, cols), 0)
    c1 = lax.broadcasted_iota(jnp.int32, (cols, cols), 1)
    m_up = (r1 < c1).astype(jnp.float32)
    r2 = lax.broadcasted_iota(jnp.int32, (rows, rows), 0)
    c2 = lax.broadcasted_iota(jnp.int32, (rows, rows), 1)
    s_lo = (c2 < r2).astype(jnp.float32)

    lane1d = lax.broadcasted_iota(jnp.int32, (LANES,), 0)
    dest = jnp.zeros(ids.shape, jnp.int32)
    gs = jnp.zeros((LANES,), jnp.int32)
    starts = []
    start = jnp.int32(0)
    for e in range(E):
        m = (ids == e).astype(jnp.float32)
        rank_in_row = lax.dot_general(m, m_up, (((1,), (0,)), ((), ())),
                                      preferred_element_type=jnp.float32)
        row_tot = jnp.sum(m, axis=1, keepdims=True)
        row_pre = lax.dot_general(s_lo, row_tot, (((1,), (0,)), ((), ())),
                                  preferred_element_type=jnp.float32)
        rank = (rank_in_row + row_pre).astype(jnp.int32)
        tot = jnp.sum(row_tot).astype(jnp.int32)
        dest = jnp.where(ids == e, start + rank, dest)
        gs = jnp.where(lane1d == e, tot, gs)
        starts.append(start // BT)
        nblk = (tot + BT - 1) // BT
        start = start + nblk * BT
    used = start // BT
    bexp = jnp.zeros((LANES,), jnp.int32)
    for e in range(1, E):
        bexp = bexp + jnp.where(
            (lane1d >= starts[e]) & (starts[e] < used), 1, 0)
    dest_ref[...] = dest
    gs_ref[...] = gs
    bexp_ref[...] = bexp
    meta_ref[...] = jnp.full((LANES,), used, jnp.int32)


def _dispatch(ids2d):
    return pl.pallas_call(
        _dispatch_body,
        grid=(1,),
        in_specs=[pl.BlockSpec((NT // LANES, LANES), lambda i: (0, 0))],
        out_specs=[
            pl.BlockSpec((NT // LANES, LANES), lambda i: (0, 0)),
            pl.BlockSpec((LANES,), lambda i: (0,)),
            pl.BlockSpec((LANES,), lambda i: (0,)),
            pl.BlockSpec((LANES,), lambda i: (0,)),
        ],
        out_shape=[
            jax.ShapeDtypeStruct((NT // LANES, LANES), jnp.int32),
            jax.ShapeDtypeStruct((LANES,), jnp.int32),
            jax.ShapeDtypeStruct((LANES,), jnp.int32),
            jax.ShapeDtypeStruct((LANES,), jnp.int32),
        ],
    )(ids2d)


def _sc_wid():
    return lax.axis_index("s") * 2 + lax.axis_index("c")


def _disperse_body(x_hbm, dest_hbm, xs_hbm, idx_v, rows_v, sem):
    wid = _sc_wid()
    pltpu.sync_copy(dest_hbm.at[wid], idx_v)
    for c in range(NCHUNK):
        pltpu.sync_copy(x_hbm.at[pl.ds(wid * TPW + c * RC, RC)], rows_v)
        pltpu.async_copy(rows_v, xs_hbm.at[idx_v.at[c]], sem).wait()


def _disperse(x2d, dest3):
    mesh = plsc.VectorSubcoreMesh(core_axis_name="c", subcore_axis_name="s")
    f = pl.kernel(
        _disperse_body,
        out_type=jax.ShapeDtypeStruct((NPAD, H), jnp.float32),
        mesh=mesh,
        scratch_types=[
            pltpu.VMEM((NCHUNK, RC), jnp.int32),
            pltpu.VMEM((RC, H), jnp.float32),
            pltpu.SemaphoreType.DMA,
        ],
    )
    return f(x2d, dest3)


def _ffn_body(bexp_ref, used_ref, x_ref, w1_ref, w2_ref, o_ref):
    del bexp_ref

    @pl.when(pl.program_id(0) < used_ref[0])
    def _active():
        xb = x_ref[...]
        nf = 8
        bf = F // nf
        for f in range(nf):
            w1b = w1_ref[0, :, f * bf:(f + 1) * bf]
            h = jnp.dot(xb, w1b, preferred_element_type=jnp.float32)
            h = 0.5 * h * (1.0 + lax.erf(h * 0.7071067811865476))
            w2b = w2_ref[0, f * bf:(f + 1) * bf, :]
            part = jnp.dot(h, w2b, preferred_element_type=jnp.float32)
            if f == 0:
                o_ref[...] = part
            else:
                o_ref[...] += part


def _ffn(bexp, used, xs, w1, w2):
    grid_spec = pltpu.PrefetchScalarGridSpec(
        num_scalar_prefetch=2,
        grid=(NB,),
        in_specs=[
            pl.BlockSpec((BT, H), lambda b, be, u: (b, 0)),
            pl.BlockSpec((1, H, F), lambda b, be, u: (be[b], 0, 0),
                         pipeline_mode=pl.Buffered(buffer_count=1)),
            pl.BlockSpec((1, F, H), lambda b, be, u: (be[b], 0, 0),
                         pipeline_mode=pl.Buffered(buffer_count=2)),
        ],
        out_specs=pl.BlockSpec((BT, H), lambda b, be, u: (b, 0)),
    )
    return pl.pallas_call(
        _ffn_body,
        grid_spec=grid_spec,
        out_shape=jax.ShapeDtypeStruct((NPAD, H), jnp.float32),
        compiler_params=pltpu.CompilerParams(
            dimension_semantics=("arbitrary",),
            vmem_limit_bytes=120 * 1024 * 1024,
        ),
    )(bexp, used, xs, w1, w2)


def _combine_body(os_hbm, dest_hbm, out_hbm, idx_v, rows_v, sem):
    wid = _sc_wid()
    pltpu.sync_copy(dest_hbm.at[wid], idx_v)
    for c in range(NCHUNK):
        pltpu.async_copy(os_hbm.at[idx_v.at[c]], rows_v, sem).wait()
        pltpu.sync_copy(rows_v, out_hbm.at[pl.ds(wid * TPW + c * RC, RC)])


def _combine(os_, dest3):
    mesh = plsc.VectorSubcoreMesh(core_axis_name="c", subcore_axis_name="s")
    f = pl.kernel(
        _combine_body,
        out_type=jax.ShapeDtypeStruct((NT, H), jnp.float32),
        mesh=mesh,
        scratch_types=[
            pltpu.VMEM((NCHUNK, RC), jnp.int32),
            pltpu.VMEM((RC, H), jnp.float32),
            pltpu.SemaphoreType.DMA,
        ],
    )
    return f(os_, dest3)


def kernel(x, gate_w, w1, w2):
    b, s, h = x.shape
    x2d = x.reshape(NT, H)
    gw_pad = jnp.zeros((LANES, H), jnp.float32).at[:E].set(gate_w)
    ids = _router(x2d, gw_pad)
    dest2d, gs, bexp, meta = _dispatch(ids.reshape(NT // LANES, LANES))
    nused = meta[:1]
    dest3 = dest2d.reshape(NW, NCHUNK, RC)
    xs = _disperse(x2d, dest3)
    os_ = _ffn(bexp[:NB], nused, xs, w1, w2)
    out2d = _combine(os_, dest3)
    return out2d.reshape(b, s, h), gs[:E]


# speedup vs baseline: 1.1357x; 1.1357x over previous
import jax
import jax.numpy as jnp
from jax import lax
from jax.experimental import pallas as pl
from jax.experimental.pallas import tpu as pltpu
from jax.experimental.pallas import tpu_sc as plsc

H = 1024
F = 4096
E = 8
NT = 4096
BT = 256
NB = NT // BT + E
NPAD = NB * BT
LANES = 128

NW = 32
TPW = NT // NW
RC = 32
NCHUNK = TPW // RC


def _router_body(x_ref, gw_ref, ids_ref):
    xb = x_ref[...]
    logits = lax.dot_general(xb, gw_ref[...], (((1,), (1,)), ((), ())),
                             preferred_element_type=jnp.float32)
    lane = lax.broadcasted_iota(jnp.int32, logits.shape, 1)
    masked = jnp.where(lane < E, logits, -jnp.inf)
    mx = jnp.max(masked, axis=1, keepdims=True)
    cand = jnp.where(masked == mx, lane, LANES)
    ids_ref[...] = jnp.min(cand, axis=1).astype(jnp.int32)


def _router(x2d, gw_pad):
    brt = 512
    return pl.pallas_call(
        _router_body,
        grid=(NT // brt,),
        in_specs=[
            pl.BlockSpec((brt, H), lambda b: (b, 0)),
            pl.BlockSpec((LANES, H), lambda b: (0, 0)),
        ],
        out_specs=pl.BlockSpec((brt,), lambda b: (b,)),
        out_shape=jax.ShapeDtypeStruct((NT,), jnp.int32),
    )(x2d, gw_pad)


def _dispatch_body(ids_ref, dest_ref, gs_ref, bexp_ref, meta_ref):
    ids = ids_ref[...]
    rows, cols = ids.shape
    r1 = lax.broadcasted_iota(jnp.int32, (cols, cols), 0)
    c1 = lax.broadcasted_iota(jnp.int32, (cols, cols), 1)
    m_up = (r1 < c1).astype(jnp.float32)
    r2 = lax.broadcasted_iota(jnp.int32, (rows, rows), 0)
    c2 = lax.broadcasted_iota(jnp.int32, (rows, rows), 1)
    s_lo = (c2 < r2).astype(jnp.float32)

    lane1d = lax.broadcasted_iota(jnp.int32, (LANES,), 0)
    dest = jnp.zeros(ids.shape, jnp.int32)
    gs = jnp.zeros((LANES,), jnp.int32)
    starts = []
    start = jnp.int32(0)
    for e in range(E):
        m = (ids == e).astype(jnp.float32)
        rank_in_row = lax.dot_general(m, m_up, (((1,), (0,)), ((), ())),
                                      preferred_element_type=jnp.float32)
        row_tot = jnp.sum(m, axis=1, keepdims=True)
        row_pre = lax.dot_general(s_lo, row_tot, (((1,), (0,)), ((), ())),
                                  preferred_element_type=jnp.float32)
        rank = (rank_in_row + row_pre).astype(jnp.int32)
        tot = jnp.sum(row_tot).astype(jnp.int32)
        dest = jnp.where(ids == e, start + rank, dest)
        gs = jnp.where(lane1d == e, tot, gs)
        starts.append(start // BT)
        nblk = (tot + BT - 1) // BT
        start = start + nblk * BT
    used = start // BT
    bexp = jnp.zeros((LANES,), jnp.int32)
    for e in range(1, E):
        bexp = bexp + jnp.where(
            (lane1d >= starts[e]) & (starts[e] < used), 1, 0)
    dest_ref[...] = dest
    gs_ref[...] = gs
    bexp_ref[...] = bexp
    meta_ref[...] = jnp.full((LANES,), used, jnp.int32)


def _dispatch(ids2d):
    return pl.pallas_call(
        _dispatch_body,
        grid=(1,),
        in_specs=[pl.BlockSpec((NT // LANES, LANES), lambda i: (0, 0))],
        out_specs=[
            pl.BlockSpec((NT // LANES, LANES), lambda i: (0, 0)),
            pl.BlockSpec((LANES,), lambda i: (0,)),
            pl.BlockSpec((LANES,), lambda i: (0,)),
            pl.BlockSpec((LANES,), lambda i: (0,)),
        ],
        out_shape=[
            jax.ShapeDtypeStruct((NT // LANES, LANES), jnp.int32),
            jax.ShapeDtypeStruct((LANES,), jnp.int32),
            jax.ShapeDtypeStruct((LANES,), jnp.int32),
            jax.ShapeDtypeStruct((LANES,), jnp.int32),
        ],
    )(ids2d)


def _sc_wid():
    return lax.axis_index("s") * 2 + lax.axis_index("c")


def _disperse_body(x_hbm, dest_hbm, xs_hbm, idx_v, rows_v,
                   l0, l1, s0, s1):
    wid = _sc_wid()
    pltpu.sync_copy(dest_hbm.at[wid], idx_v)
    lsem = [l0, l1]
    ssem = [s0, s1]
    loads = {0: pltpu.async_copy(x_hbm.at[pl.ds(wid * TPW, RC)],
                                 rows_v.at[0], lsem[0])}
    scats = [None, None]
    for c in range(NCHUNK):
        i = c % 2
        if c + 1 < NCHUNK:
            j = (c + 1) % 2
            if scats[j] is not None:
                scats[j].wait()
            loads[c + 1] = pltpu.async_copy(
                x_hbm.at[pl.ds(wid * TPW + (c + 1) * RC, RC)],
                rows_v.at[j], lsem[j])
        loads[c].wait()
        scats[i] = pltpu.async_copy(rows_v.at[i], xs_hbm.at[idx_v.at[c]],
                                    ssem[i])
    for h in scats:
        if h is not None:
            h.wait()


def _disperse(x2d, dest3):
    mesh = plsc.VectorSubcoreMesh(core_axis_name="c", subcore_axis_name="s")
    f = pl.kernel(
        _disperse_body,
        out_type=jax.ShapeDtypeStruct((NPAD, H), jnp.float32),
        mesh=mesh,
        scratch_types=[
            pltpu.VMEM((NCHUNK, RC), jnp.int32),
            pltpu.VMEM((2, RC, H), jnp.float32),
            pltpu.SemaphoreType.DMA,
            pltpu.SemaphoreType.DMA,
            pltpu.SemaphoreType.DMA,
            pltpu.SemaphoreType.DMA,
        ],
    )
    return f(x2d, dest3)


def _ffn_body(bexp_ref, used_ref, x_ref, w1_ref, w2_ref, o_ref):
    del bexp_ref

    @pl.when(pl.program_id(0) < used_ref[0])
    def _active():
        xb = x_ref[...]
        nf = 8
        bf = F // nf
        for f in range(nf):
            w1b = w1_ref[0, :, f * bf:(f + 1) * bf]
            h = jnp.dot(xb, w1b, preferred_element_type=jnp.float32)
            h = 0.5 * h * (1.0 + lax.erf(h * 0.7071067811865476))
            w2b = w2_ref[0, f * bf:(f + 1) * bf, :]
            part = jnp.dot(h, w2b, preferred_element_type=jnp.float32)
            if f == 0:
                o_ref[...] = part
            else:
                o_ref[...] += part


def _ffn(bexp, used, xs, w1, w2):
    grid_spec = pltpu.PrefetchScalarGridSpec(
        num_scalar_prefetch=2,
        grid=(NB,),
        in_specs=[
            pl.BlockSpec((BT, H), lambda b, be, u: (b, 0)),
            pl.BlockSpec((1, H, F), lambda b, be, u: (be[b], 0, 0),
                         pipeline_mode=pl.Buffered(buffer_count=1)),
            pl.BlockSpec((1, F, H), lambda b, be, u: (be[b], 0, 0),
                         pipeline_mode=pl.Buffered(buffer_count=2)),
        ],
        out_specs=pl.BlockSpec((BT, H), lambda b, be, u: (b, 0)),
    )
    return pl.pallas_call(
        _ffn_body,
        grid_spec=grid_spec,
        out_shape=jax.ShapeDtypeStruct((NPAD, H), jnp.float32),
        compiler_params=pltpu.CompilerParams(
            dimension_semantics=("arbitrary",),
            vmem_limit_bytes=120 * 1024 * 1024,
        ),
    )(bexp, used, xs, w1, w2)


def _combine_body(os_hbm, dest_hbm, out_hbm, idx_v, rows_v,
                  g0, g1, p0, p1):
    wid = _sc_wid()
    pltpu.sync_copy(dest_hbm.at[wid], idx_v)
    gsem = [g0, g1]
    psem = [p0, p1]
    gets = {0: pltpu.async_copy(os_hbm.at[idx_v.at[0]], rows_v.at[0],
                                gsem[0])}
    puts = [None, None]
    for c in range(NCHUNK):
        i = c % 2
        if c + 1 < NCHUNK:
            j = (c + 1) % 2
            if puts[j] is not None:
                puts[j].wait()
            gets[c + 1] = pltpu.async_copy(os_hbm.at[idx_v.at[c + 1]],
                                           rows_v.at[j], gsem[j])
        gets[c].wait()
        puts[i] = pltpu.async_copy(
            rows_v.at[i], out_hbm.at[pl.ds(wid * TPW + c * RC, RC)], psem[i])
    for h in puts:
        if h is not None:
            h.wait()


def _combine(os_, dest3):
    mesh = plsc.VectorSubcoreMesh(core_axis_name="c", subcore_axis_name="s")
    f = pl.kernel(
        _combine_body,
        out_type=jax.ShapeDtypeStruct((NT, H), jnp.float32),
        mesh=mesh,
        scratch_types=[
            pltpu.VMEM((NCHUNK, RC), jnp.int32),
            pltpu.VMEM((2, RC, H), jnp.float32),
            pltpu.SemaphoreType.DMA,
            pltpu.SemaphoreType.DMA,
            pltpu.SemaphoreType.DMA,
            pltpu.SemaphoreType.DMA,
        ],
    )
    return f(os_, dest3)


def kernel(x, gate_w, w1, w2):
    b, s, h = x.shape
    x2d = x.reshape(NT, H)
    gw_pad = jnp.zeros((LANES, H), jnp.float32).at[:E].set(gate_w)
    ids = _router(x2d, gw_pad)
    dest2d, gs, bexp, meta = _dispatch(ids.reshape(NT // LANES, LANES))
    nused = meta[:1]
    dest3 = dest2d.reshape(NW, NCHUNK, RC)
    xs = _disperse(x2d, dest3)
    os_ = _ffn(bexp[:NB], nused, xs, w1, w2)
    out2d = _combine(os_, dest3)
    return out2d.reshape(b, s, h), gs[:E]
